# Initial kernel scaffold; baseline (speedup 1.0000x reference)
import jax
import jax.numpy as jnp
from jax.experimental import pallas as pl


def kernel(indices, token_table, pos_table):
    raise NotImplementedError("write your pallas kernel here")


# speedup vs baseline: 1.4016x; 1.4016x over previous
import functools

import jax
import jax.numpy as jnp
from jax import lax
from jax.experimental import pallas as pl
from jax.experimental.pallas import tpu as pltpu
from jax.experimental.pallas import tpu_sc as plsc

NUM_CORES = 2
NUM_SUBCORES = 16
NUM_WORKERS = NUM_CORES * NUM_SUBCORES
LANES = 16

SEQS_PER_CHUNK = 4


def _make_lookup(B, S, D):
    assert B % NUM_WORKERS == 0
    seqs_per_worker = B // NUM_WORKERS
    assert seqs_per_worker % SEQS_PER_CHUNK == 0
    chunks = seqs_per_worker // SEQS_PER_CHUNK
    chunk_rows = SEQS_PER_CHUNK * S
    rows_per_worker = seqs_per_worker * S
    assert D == 2 * LANES
    assert chunk_rows % 8 == 0 and rows_per_worker % 8 == 0

    sub = []
    off = 0
    while off < chunk_rows:
        sz = min(128, chunk_rows - off)
        sub.append((off, sz))
        off += sz

    mesh = plsc.VectorSubcoreMesh(core_axis_name="c", subcore_axis_name="s")

    @functools.partial(
        pl.kernel,
        mesh=mesh,
        compiler_params=pltpu.CompilerParams(use_tc_tiling_on_sc=False),
        out_type=jax.ShapeDtypeStruct((B * S, D), jnp.float32),
        scratch_types=[
            pltpu.VMEM((chunk_rows,), jnp.int32),
            pltpu.VMEM((chunk_rows, D), jnp.float32),
            pltpu.VMEM((S, D), jnp.float32),
            pltpu.SemaphoreType.DMA,
        ],
    )
    def lookup(table_hbm, idx_hbm, pos_hbm, out_hbm, idx_v, rows_v, pos_v, sem):
        wid = lax.axis_index("s") * NUM_CORES + lax.axis_index("c")
        worker_base = wid * rows_per_worker

        pltpu.sync_copy(pos_hbm, pos_v)

        def chunk_body(c, carry):
            base = pl.multiple_of(worker_base + c * chunk_rows, 8)
            pltpu.sync_copy(idx_hbm.at[pl.ds(base, chunk_rows)], idx_v)
            copies = []
            for (o, sz) in sub:
                copies.append(
                    pltpu.make_async_copy(
                        table_hbm.at[idx_v.at[pl.ds(o, sz)]],
                        rows_v.at[pl.ds(o, sz)],
                        sem,
                    )
                )
            for cp in copies:
                cp.start()
            for cp in copies:
                cp.wait()

            def add_body(s, carry2):
                p0 = pos_v[s, pl.ds(0, LANES)]
                p1 = pos_v[s, pl.ds(LANES, LANES)]
                for q in range(SEQS_PER_CHUNK):
                    r = q * S + s
                    rows_v[r, pl.ds(0, LANES)] += p0
                    rows_v[r, pl.ds(LANES, LANES)] += p1
                return carry2

            lax.fori_loop(0, S, add_body, 0)
            pltpu.sync_copy(rows_v, out_hbm.at[pl.ds(base, chunk_rows)])
            return carry

        lax.fori_loop(0, chunks, chunk_body, 0)

    return lookup


def kernel(indices, token_table, pos_table):
    B, S = indices.shape
    V, D = token_table.shape
    idx_flat = indices.reshape(B * S).astype(jnp.int32)
    pos_rows = lax.slice(pos_table, (0, 0), (S, D))
    lookup = _make_lookup(B, S, D)
    out = lookup(token_table, idx_flat, pos_rows)
    return out.reshape(B, S, D)
